# speedup vs baseline: 7.1104x; 7.1104x over previous
import functools

import jax
import jax.numpy as jnp
from jax.experimental import pallas as pl
from jax.experimental.pallas import tpu as pltpu

B, T, IN, H = 64, 128, 1024, 2048
FH = 4 * H
TB = T * B
BH = B // 2


def _gemm_bias_kernel(x_ref, w_ref, b_ref, o_ref):
    o_ref[...] = (
        jnp.dot(x_ref[...], w_ref[...], preferred_element_type=jnp.float32)
        + b_ref[...]
    )


def _lstm_step_kernel(yx_ref, wat_ref, a0_ref, c0_ref,
                      a_out, c_out, yi_out, yf_out, yg_out, yo_out,
                      a_scr, c_scr):
    t = pl.program_id(1)

    @pl.when(t == 0)
    def _init():
        a_scr[...] = a0_ref[...]
        c_scr[...] = c0_ref[...]

    a_prev = a_scr[...]
    c_prev = c_scr[...]
    y = yx_ref[...] + jnp.dot(
        a_prev.astype(jnp.bfloat16), wat_ref[...],
        preferred_element_type=jnp.float32)
    yi = y[:, 0 * H:1 * H]
    yf = y[:, 1 * H:2 * H]
    yg = y[:, 2 * H:3 * H]
    yo = y[:, 3 * H:4 * H]
    c_t = jax.nn.sigmoid(yf) * c_prev + jax.nn.sigmoid(yi) * jnp.tanh(yg)
    a_t = jax.nn.sigmoid(yo) * jnp.tanh(c_t)
    yi_out[...] = yi
    yf_out[...] = yf
    yg_out[...] = yg
    yo_out[...] = yo
    c_out[...] = c_t
    a_out[...] = a_t
    a_scr[...] = a_t
    c_scr[...] = c_t


@jax.jit
def kernel(x, Wx, Wa, b, a0, c0):
    x_tm = jnp.swapaxes(x, 0, 1).reshape(TB, IN).astype(jnp.bfloat16)
    wxt = Wx.T.astype(jnp.bfloat16)
    b2 = b.reshape(1, FH)

    BM, BN = 1024, 1024
    yx = pl.pallas_call(
        _gemm_bias_kernel,
        grid=(TB // BM, FH // BN),
        in_specs=[
            pl.BlockSpec((BM, IN), lambda i, j: (i, 0)),
            pl.BlockSpec((IN, BN), lambda i, j: (0, j)),
            pl.BlockSpec((1, BN), lambda i, j: (0, j)),
        ],
        out_specs=pl.BlockSpec((BM, BN), lambda i, j: (i, j)),
        out_shape=jax.ShapeDtypeStruct((TB, FH), jnp.float32),
        compiler_params=pltpu.CompilerParams(
            dimension_semantics=("parallel", "arbitrary")),
    )(x_tm, wxt, b2)

    wat = Wa.T.astype(jnp.bfloat16)

    out_sd = jax.ShapeDtypeStruct((B, T * H), jnp.float32)
    out_spec = pl.BlockSpec((BH, H), lambda h, t: (h, t))
    outs = pl.pallas_call(
        _lstm_step_kernel,
        grid=(2, T),
        in_specs=[
            pl.BlockSpec((BH, FH), lambda h, t: (2 * t + h, 0)),
            pl.BlockSpec((H, FH), lambda h, t: (0, 0)),
            pl.BlockSpec((BH, H), lambda h, t: (h, 0)),
            pl.BlockSpec((BH, H), lambda h, t: (h, 0)),
        ],
        out_specs=[out_spec] * 6,
        out_shape=[out_sd] * 6,
        scratch_shapes=[
            pltpu.VMEM((BH, H), jnp.float32),
            pltpu.VMEM((BH, H), jnp.float32),
        ],
        compiler_params=pltpu.CompilerParams(
            dimension_semantics=("parallel", "arbitrary")),
    )(yx, wat, a0, c0)

    a, c, yi, yf, yg, yo = (o.reshape(B, T, H) for o in outs)
    return (a, c, yi, yf, yg, yo)


# speedup vs baseline: 10.9136x; 1.5349x over previous
import functools

import jax
import jax.numpy as jnp
from jax.experimental import pallas as pl
from jax.experimental.pallas import tpu as pltpu

B, T, IN, H = 64, 128, 1024, 2048
FH = 4 * H
TB = T * B
BH = B // 2


def _gemm_bias_kernel(x_ref, w_ref, b_ref, o_ref):
    o_ref[...] = (
        jnp.dot(x_ref[...], w_ref[...], preferred_element_type=jnp.float32)
        + b_ref[...]
    )


def _lstm_step_kernel(yx_ref, wat_ref, a0_ref, c0_ref,
                      a_out, c_out, yi_out, yf_out, yg_out, yo_out,
                      a_scr, c_scr):
    t = pl.program_id(0)

    @pl.when(t == 0)
    def _init():
        a_scr[...] = a0_ref[...]
        c_scr[...] = c0_ref[...]

    a_prev = a_scr[...]
    c_prev = c_scr[...]
    y = yx_ref[...] + jnp.dot(
        a_prev.astype(jnp.bfloat16), wat_ref[...],
        preferred_element_type=jnp.float32)
    yi = y[:, 0 * H:1 * H]
    yf = y[:, 1 * H:2 * H]
    yg = y[:, 2 * H:3 * H]
    yo = y[:, 3 * H:4 * H]
    c_t = jax.nn.sigmoid(yf) * c_prev + jax.nn.sigmoid(yi) * jnp.tanh(yg)
    a_t = jax.nn.sigmoid(yo) * jnp.tanh(c_t)
    yi_out[...] = yi
    yf_out[...] = yf
    yg_out[...] = yg
    yo_out[...] = yo
    c_out[...] = c_t
    a_out[...] = a_t
    a_scr[...] = a_t
    c_scr[...] = c_t


@jax.jit
def kernel(x, Wx, Wa, b, a0, c0):
    x_tm = jnp.swapaxes(x, 0, 1).reshape(TB, IN).astype(jnp.bfloat16)
    wxt = Wx.T.astype(jnp.bfloat16)
    b2 = b.reshape(1, FH)

    BM, BN = 1024, 1024
    yx = pl.pallas_call(
        _gemm_bias_kernel,
        grid=(TB // BM, FH // BN),
        in_specs=[
            pl.BlockSpec((BM, IN), lambda i, j: (i, 0)),
            pl.BlockSpec((IN, BN), lambda i, j: (0, j)),
            pl.BlockSpec((1, BN), lambda i, j: (0, j)),
        ],
        out_specs=pl.BlockSpec((BM, BN), lambda i, j: (i, j)),
        out_shape=jax.ShapeDtypeStruct((TB, FH), jnp.float32),
        compiler_params=pltpu.CompilerParams(
            dimension_semantics=("parallel", "arbitrary")),
    )(x_tm, wxt, b2)

    wat = Wa.T.astype(jnp.bfloat16)

    out_sd = jax.ShapeDtypeStruct((B, T * H), jnp.float32)
    out_spec = pl.BlockSpec((B, H), lambda t: (0, t))
    outs = pl.pallas_call(
        _lstm_step_kernel,
        grid=(T,),
        in_specs=[
            pl.BlockSpec((B, FH), lambda t: (t, 0)),
            pl.BlockSpec((H, FH), lambda t: (0, 0)),
            pl.BlockSpec((B, H), lambda t: (0, 0)),
            pl.BlockSpec((B, H), lambda t: (0, 0)),
        ],
        out_specs=[out_spec] * 6,
        out_shape=[out_sd] * 6,
        scratch_shapes=[
            pltpu.VMEM((B, H), jnp.float32),
            pltpu.VMEM((B, H), jnp.float32),
        ],
        compiler_params=pltpu.CompilerParams(
            dimension_semantics=("arbitrary",)),
    )(yx, wat, a0, c0)

    a, c, yi, yf, yg, yo = (o.reshape(B, T, H) for o in outs)
    return (a, c, yi, yf, yg, yo)


# speedup vs baseline: 11.0520x; 1.0127x over previous
import functools

import jax
import jax.numpy as jnp
from jax.experimental import pallas as pl
from jax.experimental.pallas import tpu as pltpu

B, T, IN, H = 64, 128, 1024, 2048
FH = 4 * H
TB = T * B
BH = B // 2


def _gemm_bias_kernel(x_ref, w_ref, b_ref, o_ref):
    o_ref[...] = (
        jnp.dot(x_ref[...], w_ref[...], preferred_element_type=jnp.float32)
        + b_ref[...]
    ).astype(jnp.bfloat16)


def _lstm_step_kernel(yx_ref, wat_ref, a0_ref, c0_ref,
                      a_out, c_out, yi_out, yf_out, yg_out, yo_out,
                      a_scr, c_scr):
    t = pl.program_id(0)

    @pl.when(t == 0)
    def _init():
        a_scr[...] = a0_ref[...]
        c_scr[...] = c0_ref[...]

    a_prev = a_scr[...]
    c_prev = c_scr[...]
    y = yx_ref[...] + jnp.dot(
        a_prev.astype(jnp.bfloat16), wat_ref[...],
        preferred_element_type=jnp.float32)
    yi = y[:, 0 * H:1 * H]
    yf = y[:, 1 * H:2 * H]
    yg = y[:, 2 * H:3 * H]
    yo = y[:, 3 * H:4 * H]
    c_t = jax.nn.sigmoid(yf) * c_prev + jax.nn.sigmoid(yi) * jnp.tanh(yg)
    a_t = jax.nn.sigmoid(yo) * jnp.tanh(c_t)
    yi_out[...] = yi
    yf_out[...] = yf
    yg_out[...] = yg
    yo_out[...] = yo
    c_out[...] = c_t
    a_out[...] = a_t
    a_scr[...] = a_t
    c_scr[...] = c_t


@jax.jit
def kernel(x, Wx, Wa, b, a0, c0):
    x_tm = jnp.swapaxes(x, 0, 1).reshape(TB, IN).astype(jnp.bfloat16)
    wxt = Wx.T.astype(jnp.bfloat16)
    b2 = b.reshape(1, FH)

    BM, BN = 2048, 1024
    yx = pl.pallas_call(
        _gemm_bias_kernel,
        grid=(TB // BM, FH // BN),
        in_specs=[
            pl.BlockSpec((BM, IN), lambda i, j: (i, 0)),
            pl.BlockSpec((IN, BN), lambda i, j: (0, j)),
            pl.BlockSpec((1, BN), lambda i, j: (0, j)),
        ],
        out_specs=pl.BlockSpec((BM, BN), lambda i, j: (i, j)),
        out_shape=jax.ShapeDtypeStruct((TB, FH), jnp.bfloat16),
        compiler_params=pltpu.CompilerParams(
            dimension_semantics=("parallel", "arbitrary")),
    )(x_tm, wxt, b2)

    wat = Wa.T.astype(jnp.bfloat16)

    out_sd = jax.ShapeDtypeStruct((B, T * H), jnp.float32)
    out_spec = pl.BlockSpec((B, H), lambda t: (0, t))
    outs = pl.pallas_call(
        _lstm_step_kernel,
        grid=(T,),
        in_specs=[
            pl.BlockSpec((B, FH), lambda t: (t, 0)),
            pl.BlockSpec((H, FH), lambda t: (0, 0)),
            pl.BlockSpec((B, H), lambda t: (0, 0)),
            pl.BlockSpec((B, H), lambda t: (0, 0)),
        ],
        out_specs=[out_spec] * 6,
        out_shape=[out_sd] * 6,
        scratch_shapes=[
            pltpu.VMEM((B, H), jnp.float32),
            pltpu.VMEM((B, H), jnp.float32),
        ],
        compiler_params=pltpu.CompilerParams(
            dimension_semantics=("arbitrary",)),
    )(yx, wat, a0, c0)

    a, c, yi, yf, yg, yo = (o.reshape(B, T, H) for o in outs)
    return (a, c, yi, yf, yg, yo)


# speedup vs baseline: 11.1389x; 1.0079x over previous
import functools

import jax
import jax.numpy as jnp
from jax.experimental import pallas as pl
from jax.experimental.pallas import tpu as pltpu

B, T, IN, H = 64, 128, 1024, 2048
FH = 4 * H
TB = T * B
BH = B // 2


def _gemm_bias_kernel(x_ref, w_ref, b_ref, o_ref):
    o_ref[...] = (
        jnp.dot(x_ref[...], w_ref[...], preferred_element_type=jnp.float32)
        + b_ref[...]
    ).astype(jnp.bfloat16)


U = 2


def _lstm_step_kernel(yx_ref, wat_ref, a0_ref, c0_ref,
                      a_out, c_out, yi_out, yf_out, yg_out, yo_out,
                      a_scr, c_scr):
    t = pl.program_id(0)

    @pl.when(t == 0)
    def _init():
        a_scr[...] = a0_ref[...]
        c_scr[...] = c0_ref[...]

    a_prev = a_scr[...]
    c_prev = c_scr[...]
    for s in range(U):
        y = yx_ref[s * B:(s + 1) * B, :] + jnp.dot(
            a_prev.astype(jnp.bfloat16), wat_ref[...],
            preferred_element_type=jnp.float32)
        yi = y[:, 0 * H:1 * H]
        yf = y[:, 1 * H:2 * H]
        yg = y[:, 2 * H:3 * H]
        yo = y[:, 3 * H:4 * H]
        c_t = jax.nn.sigmoid(yf) * c_prev + jax.nn.sigmoid(yi) * jnp.tanh(yg)
        a_t = jax.nn.sigmoid(yo) * jnp.tanh(c_t)
        hs = slice(s * H, (s + 1) * H)
        yi_out[:, hs] = yi
        yf_out[:, hs] = yf
        yg_out[:, hs] = yg
        yo_out[:, hs] = yo
        c_out[:, hs] = c_t
        a_out[:, hs] = a_t
        a_prev, c_prev = a_t, c_t
    a_scr[...] = a_prev
    c_scr[...] = c_prev


@jax.jit
def kernel(x, Wx, Wa, b, a0, c0):
    x_tm = jnp.swapaxes(x, 0, 1).reshape(TB, IN).astype(jnp.bfloat16)
    wxt = Wx.T.astype(jnp.bfloat16)
    b2 = b.reshape(1, FH)

    BM, BN = 2048, 1024
    yx = pl.pallas_call(
        _gemm_bias_kernel,
        grid=(TB // BM, FH // BN),
        in_specs=[
            pl.BlockSpec((BM, IN), lambda i, j: (i, 0)),
            pl.BlockSpec((IN, BN), lambda i, j: (0, j)),
            pl.BlockSpec((1, BN), lambda i, j: (0, j)),
        ],
        out_specs=pl.BlockSpec((BM, BN), lambda i, j: (i, j)),
        out_shape=jax.ShapeDtypeStruct((TB, FH), jnp.bfloat16),
        compiler_params=pltpu.CompilerParams(
            dimension_semantics=("parallel", "arbitrary")),
    )(x_tm, wxt, b2)

    wat = Wa.T.astype(jnp.bfloat16)

    out_sd = jax.ShapeDtypeStruct((B, T * H), jnp.float32)
    out_spec = pl.BlockSpec((B, U * H), lambda t: (0, t))
    outs = pl.pallas_call(
        _lstm_step_kernel,
        grid=(T // U,),
        in_specs=[
            pl.BlockSpec((U * B, FH), lambda t: (t, 0)),
            pl.BlockSpec((H, FH), lambda t: (0, 0)),
            pl.BlockSpec((B, H), lambda t: (0, 0)),
            pl.BlockSpec((B, H), lambda t: (0, 0)),
        ],
        out_specs=[out_spec] * 6,
        out_shape=[out_sd] * 6,
        scratch_shapes=[
            pltpu.VMEM((B, H), jnp.float32),
            pltpu.VMEM((B, H), jnp.float32),
        ],
        compiler_params=pltpu.CompilerParams(
            dimension_semantics=("arbitrary",)),
    )(yx, wat, a0, c0)

    a, c, yi, yf, yg, yo = (o.reshape(B, T, H) for o in outs)
    return (a, c, yi, yf, yg, yo)
